# baseline (speedup 1.0000x reference)
import functools

import jax
import jax.numpy as jnp
from jax import lax
from jax.experimental import pallas as pl
from jax.experimental.pallas import tpu as pltpu
from jax.experimental.pallas import tpu_sc as plsc

N = 10000
E = 320000
D = 128

NC = 2
NS = 16
NW = NC * NS
EW = E // NW
EP = 327680
EWP = EP // NW
K = 64
NCHUNK = EWP // K
NP = 10240
NPAD = 8
ZROWS = NP // NS
ZB = 128

_F32 = jnp.float32


def _dot(a, b):
    return jax.lax.dot_general(
        a, b, (((1,), (0,)), ((), ())),
        preferred_element_type=_F32, precision=lax.Precision.DEFAULT)


KD = 128
NCHUNK_D = (EP // NS) // KD
NIB_D = 8
IA_D = 4
WB = 64


def _fill_const(ref, nrows, vec16):
    def fill(r, carry):
        for j in range(D // 16):
            ref[r, pl.ds(j * 16, 16)] = vec16
        return carry

    lax.fori_loop(0, nrows, fill, 0)


def _zero_acc(zb_v, acc_sh, s, nrows):
    r0 = s * ZROWS
    for t in range(ZROWS // nrows):
        pltpu.sync_copy(zb_v, acc_sh.at[pl.ds(r0 + t * nrows, nrows)])


def _copy_out(acc_sh, out_hbm, c, s):
    r0 = s * ZROWS
    for t in range(ZROWS // ZB):
        pltpu.sync_copy(acc_sh.at[pl.ds(r0 + t * ZB, ZB)],
                        out_hbm.at[c, pl.ds(r0 + t * ZB, ZB)])


def _deg_body(eidx_hbm, out_hbm, idxr_v, ones_v, stage_v, wide_v, tbl_sh,
              isem, ssem):
    c = lax.axis_index("c")
    s = lax.axis_index("s")

    z16 = jnp.zeros((16,), _F32)
    for r in range(KD // 16):
        ones_v[pl.ds(r * 16, 16)] = z16
    r0 = s * ZROWS
    for t in range(ZROWS // KD):
        pltpu.sync_copy(ones_v, tbl_sh.at[pl.ds(r0 + t * KD, KD)])

    one16 = jnp.full((16,), 1.0, _F32)
    for r in range(KD // 16):
        ones_v[pl.ds(r * 16, 16)] = one16
    plsc.subcore_barrier()

    ebase = c * EP + s * (NCHUNK_D * KD)

    def istart(j, ib):
        pltpu.async_copy(eidx_hbm.at[pl.ds(ebase + j * KD, KD)],
                         idxr_v.at[ib], isem.at[ib])

    def iwait(j, ib):
        pltpu.make_async_copy(eidx_hbm.at[pl.ds(ebase + j * KD, KD)],
                              idxr_v.at[ib], isem.at[ib]).wait()

    def sstart(j, ib):
        pltpu.async_copy(ones_v, tbl_sh.at[idxr_v.at[ib]], ssem.at[ib],
                         add=True)

    def swait(j, ib):
        pltpu.make_async_copy(ones_v, tbl_sh.at[idxr_v.at[ib]],
                              ssem.at[ib]).wait()

    for j in range(IA_D):
        istart(j, j)

    def rnd(i, carry):
        for u in range(NIB_D):
            j = i * NIB_D + u
            ib4 = (u + IA_D) % NIB_D

            @pl.when(j >= IA_D)
            def _():
                swait(j - IA_D, ib4)

            @pl.when(j + IA_D < NCHUNK_D)
            def _():
                istart(j + IA_D, ib4)

            iwait(j, u)
            sstart(j, u)
        return carry

    lax.fori_loop(0, NCHUNK_D // NIB_D, rnd, 0)
    for t in range(IA_D):
        j = NCHUNK_D - IA_D + t
        swait(j, j % NIB_D)

    plsc.subcore_barrier()

    pltpu.sync_copy(tbl_sh.at[pl.ds(r0, ZROWS)], stage_v)

    for t in range(ZROWS // WB):
        def widen(g, carry):
            v = stage_v[pl.ds(t * WB + g * 16, 16)]
            for l in range(16):
                wide_v[g * 16 + l, pl.ds(0, 16)] = jnp.full((16,), v[l])
            return carry

        lax.fori_loop(0, WB // 16, widen, 0)
        pltpu.sync_copy(wide_v, out_hbm.at[c, pl.ds(r0 + t * WB, WB)])


@functools.cache
def _deg_call():
    return pl.kernel(
        _deg_body,
        out_type=jax.ShapeDtypeStruct((NC, NP, D), _F32),
        mesh=plsc.VectorSubcoreMesh(core_axis_name="c", subcore_axis_name="s"),
        scratch_types=[
            pltpu.VMEM((NIB_D, KD), jnp.int32),
            pltpu.VMEM((KD,), _F32),
            pltpu.VMEM((ZROWS,), _F32),
            pltpu.VMEM((WB, D), _F32),
            pltpu.VMEM_SHARED((NP + NPAD,), _F32),
            pltpu.SemaphoreType.DMA((NIB_D,)),
            pltpu.SemaphoreType.DMA((NIB_D,)),
        ],
    )


NBUF = 5
GA = 1
NIB = 10
IA = 4


def _agg_body(m_hbm, est_hbm, out_hbm, idxr_v, rows_v, acc_sh,
              isem, gsem, ssem):
    c = lax.axis_index("c")
    s = lax.axis_index("s")
    wid = s * NC + c

    _fill_const(rows_v.at[0], K, jnp.zeros((16,), _F32))
    _zero_acc(rows_v.at[0], acc_sh, s, K)
    plsc.subcore_barrier()

    def istart(j, ib):
        pltpu.async_copy(est_hbm.at[wid, j], idxr_v.at[ib], isem.at[ib])

    def iwait(j, ib):
        pltpu.make_async_copy(est_hbm.at[wid, j], idxr_v.at[ib],
                              isem.at[ib]).wait()

    def gstart(j, ib, b):
        pltpu.async_copy(m_hbm.at[idxr_v.at[ib, 0]], rows_v.at[b],
                         gsem.at[b])

    def gwait(j, ib, b):
        pltpu.make_async_copy(m_hbm.at[idxr_v.at[ib, 0]], rows_v.at[b],
                              gsem.at[b]).wait()

    def sstart(j, ib, b):
        pltpu.async_copy(rows_v.at[b], acc_sh.at[idxr_v.at[ib, 1]],
                         ssem.at[b], add=True)

    def swait(j, ib, b):
        pltpu.make_async_copy(rows_v.at[b], acc_sh.at[idxr_v.at[ib, 1]],
                              ssem.at[b]).wait()

    for j in range(IA):
        istart(j, j)
    for j in range(GA):
        iwait(j, j)
        gstart(j, j, j)

    def rnd(i, carry):
        for u in range(NIB):
            j = i * NIB + u
            b = u % NBUF
            ibg = (u + GA) % NIB
            bg = (u + GA) % NBUF
            ib4 = (u + IA) % NIB

            @pl.when(j + IA < NCHUNK)
            def _():
                istart(j + IA, ib4)

            @pl.when(j >= NBUF - GA)
            def _():
                swait(j - (NBUF - GA), (u - (NBUF - GA)) % NIB, bg)

            @pl.when(j + GA < NCHUNK)
            def _():
                iwait(j + GA, ibg)
                gstart(j + GA, ibg, bg)

            gwait(j, u, b)
            sstart(j, u, b)
        return carry

    lax.fori_loop(0, NCHUNK // NIB, rnd, 0)
    for t in range(NBUF - GA):
        j = NCHUNK - (NBUF - GA) + t
        swait(j, j % NIB, j % NBUF)

    plsc.subcore_barrier()
    _copy_out(acc_sh, out_hbm, c, s)


@functools.cache
def _agg_call():
    return pl.kernel(
        _agg_body,
        out_type=jax.ShapeDtypeStruct((NC, NP, D), _F32),
        mesh=plsc.VectorSubcoreMesh(core_axis_name="c", subcore_axis_name="s"),
        scratch_types=[
            pltpu.VMEM((NIB, 2, K), jnp.int32),
            pltpu.VMEM((NBUF, K, D), _F32),
            pltpu.VMEM_SHARED((NP + NPAD, D), _F32),
            pltpu.SemaphoreType.DMA((NIB,)),
            pltpu.SemaphoreType.DMA((NBUF,)),
            pltpu.SemaphoreType.DMA((NBUF,)),
        ],
    )


BN = 1000
NBLK = N // BN


def _pre_body(x_ref, we_ref, be_ref, dg_ref, w0_ref, o_ref):
    h0 = _dot(x_ref[...], we_ref[...]) + be_ref[...]
    do = jnp.maximum(dg_ref[0], 1.0)
    o_ref[...] = _dot(h0 * lax.rsqrt(do), w0_ref[...])


def _mid_body(p_ref, dg_ref, b_ref, w_ref, o_ref):
    agg = p_ref[0] + p_ref[1]
    di = jnp.maximum(dg_ref[1], 1.0)
    do = jnp.maximum(dg_ref[0], 1.0)
    h = jnp.maximum(agg * lax.rsqrt(di) + b_ref[...], 0.0)
    hn = h * lax.rsqrt(do)
    o_ref[...] = _dot(hn, w_ref[...])


def _final_body(p_ref, dg_ref, b_ref, wo1_ref, bo1_ref, wo2_ref, bo2_ref,
                o_ref):
    agg = p_ref[0] + p_ref[1]
    di = jnp.maximum(dg_ref[1], 1.0)
    h = jnp.maximum(agg * lax.rsqrt(di) + b_ref[...], 0.0)
    t = jnp.maximum(_dot(h, wo1_ref[...]) + bo1_ref[...], 0.0)
    o_ref[...] = _dot(t, wo2_ref[...]) + bo2_ref[...]


def _row_spec(width):
    return pl.BlockSpec((BN, width), lambda i: (i, 0))


def _part_spec(width):
    return pl.BlockSpec((NC, BN, width), lambda i: (0, i, 0))


def _full_spec(r, cdim):
    return pl.BlockSpec((r, cdim), lambda i: (0, 0))


_pre_call = pl.pallas_call(
    _pre_body,
    grid=(NBLK,),
    in_specs=[_row_spec(D), _full_spec(D, D), _full_spec(1, D),
              _part_spec(1), _full_spec(D, D)],
    out_specs=_row_spec(D),
    out_shape=jax.ShapeDtypeStruct((N, D), _F32),
)

_mid_call = pl.pallas_call(
    _mid_body,
    grid=(NBLK,),
    in_specs=[_part_spec(D), _part_spec(1),
              _full_spec(1, D), _full_spec(D, D)],
    out_specs=_row_spec(D),
    out_shape=jax.ShapeDtypeStruct((N, D), _F32),
)

_final_call = pl.pallas_call(
    _final_body,
    grid=(NBLK,),
    in_specs=[_part_spec(D), _part_spec(1), _full_spec(1, D),
              _full_spec(D, D), _full_spec(1, D), _full_spec(D, 1),
              _full_spec(1, 1)],
    out_specs=_row_spec(1),
    out_shape=jax.ShapeDtypeStruct((N, 1), _F32),
)


def kernel(node_feats, edge_index, W_emb, b_emb, W0, b0, W1, b1, W2, b2,
           Wo1, bo1, Wo2, bo2):
    src = edge_index[0]
    dst = edge_index[1]
    npad = EP - E
    pad_dummy = (NP + (jnp.arange(npad, dtype=jnp.int32) % NPAD))
    pad_gather = (jnp.arange(npad, dtype=jnp.int32) * 131) % N
    src_p = jnp.concatenate([src, pad_gather])
    dst_p = jnp.concatenate([dst, pad_dummy])
    est = jnp.stack([src_p.reshape(NW, NCHUNK, K),
                     dst_p.reshape(NW, NCHUNK, K)], axis=2)
    eidx = jnp.concatenate([src, pad_dummy, dst, pad_dummy])

    deg = _deg_call()(eidx)[:, :, 0:1]
    m = _pre_call(node_feats, W_emb, b_emb.reshape(1, D), deg, W0)
    p = _agg_call()(m, est)
    m = _mid_call(p, deg, b0.reshape(1, D), W1)
    p = _agg_call()(m, est)
    m = _mid_call(p, deg, b1.reshape(1, D), W2)
    p = _agg_call()(m, est)
    out = _final_call(p, deg, b2.reshape(1, D), Wo1, bo1.reshape(1, D),
                      Wo2, bo2.reshape(1, 1))
    return out


# speedup vs baseline: 1.1724x; 1.1724x over previous
import functools

import jax
import jax.numpy as jnp
from jax import lax
from jax.experimental import pallas as pl
from jax.experimental.pallas import tpu as pltpu
from jax.experimental.pallas import tpu_sc as plsc

N = 10000
E = 320000
D = 128

NC = 2
NS = 16
NW = NC * NS
EW = E // NW
EP = 327680
EWP = EP // NW
K = 64
NCHUNK = EWP // K
NP = 10240
NPAD = 8
ZROWS = NP // NS
ZB = 128

_F32 = jnp.float32


def _dot(a, b):
    return jax.lax.dot_general(
        a, b, (((1,), (0,)), ((), ())),
        preferred_element_type=_F32, precision=lax.Precision.DEFAULT)


KD = 128
NCHUNK_D = (EP // NS) // KD
NIB_D = 8
IA_D = 4
WB = 64


def _fill_const(ref, nrows, vec16):
    def fill(r, carry):
        for j in range(D // 16):
            ref[r, pl.ds(j * 16, 16)] = vec16
        return carry

    lax.fori_loop(0, nrows, fill, 0)


def _zero_acc(zb_v, acc_sh, s, nrows):
    r0 = s * ZROWS
    for t in range(ZROWS // nrows):
        pltpu.sync_copy(zb_v, acc_sh.at[pl.ds(r0 + t * nrows, nrows)])


def _copy_out(acc_sh, out_hbm, c, s):
    r0 = s * ZROWS
    for t in range(ZROWS // ZB):
        pltpu.sync_copy(acc_sh.at[pl.ds(r0 + t * ZB, ZB)],
                        out_hbm.at[c, pl.ds(r0 + t * ZB, ZB)])


def _deg_body(eidx_hbm, out_hbm, idxr_v, ones_v, stage_v, wide_v, tbl_sh,
              isem, ssem):
    c = lax.axis_index("c")
    s = lax.axis_index("s")

    z16 = jnp.zeros((16,), _F32)
    for r in range(KD // 16):
        ones_v[pl.ds(r * 16, 16)] = z16
    r0 = s * ZROWS
    for t in range(ZROWS // KD):
        pltpu.sync_copy(ones_v, tbl_sh.at[pl.ds(r0 + t * KD, KD)])

    one16 = jnp.full((16,), 1.0, _F32)
    for r in range(KD // 16):
        ones_v[pl.ds(r * 16, 16)] = one16
    plsc.subcore_barrier()

    ebase = c * EP + s * (NCHUNK_D * KD)

    def istart(j, ib):
        pltpu.async_copy(eidx_hbm.at[pl.ds(ebase + j * KD, KD)],
                         idxr_v.at[ib], isem.at[ib])

    def iwait(j, ib):
        pltpu.make_async_copy(eidx_hbm.at[pl.ds(ebase + j * KD, KD)],
                              idxr_v.at[ib], isem.at[ib]).wait()

    def sstart(j, ib):
        pltpu.async_copy(ones_v, tbl_sh.at[idxr_v.at[ib]], ssem.at[ib],
                         add=True)

    def swait(j, ib):
        pltpu.make_async_copy(ones_v, tbl_sh.at[idxr_v.at[ib]],
                              ssem.at[ib]).wait()

    for j in range(IA_D):
        istart(j, j)

    def rnd(i, carry):
        for u in range(NIB_D):
            j = i * NIB_D + u
            ib4 = (u + IA_D) % NIB_D

            @pl.when(j >= IA_D)
            def _():
                swait(j - IA_D, ib4)

            @pl.when(j + IA_D < NCHUNK_D)
            def _():
                istart(j + IA_D, ib4)

            iwait(j, u)
            sstart(j, u)
        return carry

    lax.fori_loop(0, NCHUNK_D // NIB_D, rnd, 0)
    for t in range(IA_D):
        j = NCHUNK_D - IA_D + t
        swait(j, j % NIB_D)

    plsc.subcore_barrier()

    pltpu.sync_copy(tbl_sh.at[pl.ds(r0, ZROWS)], stage_v)

    for t in range(ZROWS // WB):
        def widen(g, carry):
            v = stage_v[pl.ds(t * WB + g * 16, 16)]
            for l in range(16):
                wide_v[g * 16 + l, pl.ds(0, 16)] = jnp.full((16,), v[l])
            return carry

        lax.fori_loop(0, WB // 16, widen, 0)
        pltpu.sync_copy(wide_v, out_hbm.at[c, pl.ds(r0 + t * WB, WB)])


@functools.cache
def _deg_call():
    return pl.kernel(
        _deg_body,
        out_type=jax.ShapeDtypeStruct((NC, NP, D), _F32),
        mesh=plsc.VectorSubcoreMesh(core_axis_name="c", subcore_axis_name="s"),
        scratch_types=[
            pltpu.VMEM((NIB_D, KD), jnp.int32),
            pltpu.VMEM((KD,), _F32),
            pltpu.VMEM((ZROWS,), _F32),
            pltpu.VMEM((WB, D), _F32),
            pltpu.VMEM_SHARED((NP + NPAD,), _F32),
            pltpu.SemaphoreType.DMA((NIB_D,)),
            pltpu.SemaphoreType.DMA((NIB_D,)),
        ],
    )


NBUF = 5
GA = 3
NIB = 10
IA = 4


def _agg_body(m_hbm, est_hbm, out_hbm, idxr_v, rows_v, acc_sh,
              isem, gsem, ssem):
    c = lax.axis_index("c")
    s = lax.axis_index("s")
    wid = s * NC + c

    _fill_const(rows_v.at[0], K, jnp.zeros((16,), _F32))
    _zero_acc(rows_v.at[0], acc_sh, s, K)
    plsc.subcore_barrier()

    def istart(j, ib):
        pltpu.async_copy(est_hbm.at[wid, j], idxr_v.at[ib], isem.at[ib])

    def iwait(j, ib):
        pltpu.make_async_copy(est_hbm.at[wid, j], idxr_v.at[ib],
                              isem.at[ib]).wait()

    def gstart(j, ib, b):
        pltpu.async_copy(m_hbm.at[idxr_v.at[ib, 0]], rows_v.at[b],
                         gsem.at[b])

    def gwait(j, ib, b):
        pltpu.make_async_copy(m_hbm.at[idxr_v.at[ib, 0]], rows_v.at[b],
                              gsem.at[b]).wait()

    def sstart(j, ib, b):
        pltpu.async_copy(rows_v.at[b], acc_sh.at[idxr_v.at[ib, 1]],
                         ssem.at[b], add=True)

    def swait(j, ib, b):
        pltpu.make_async_copy(rows_v.at[b], acc_sh.at[idxr_v.at[ib, 1]],
                              ssem.at[b]).wait()

    for j in range(IA):
        istart(j, j)
    for j in range(GA):
        iwait(j, j)
        gstart(j, j, j)

    def rnd(i, carry):
        for u in range(NIB):
            j = i * NIB + u
            b = u % NBUF
            ibg = (u + GA) % NIB
            bg = (u + GA) % NBUF
            ib4 = (u + IA) % NIB

            @pl.when(j + IA < NCHUNK)
            def _():
                istart(j + IA, ib4)

            @pl.when(j >= NBUF - GA)
            def _():
                swait(j - (NBUF - GA), (u - (NBUF - GA)) % NIB, bg)

            @pl.when(j + GA < NCHUNK)
            def _():
                iwait(j + GA, ibg)
                gstart(j + GA, ibg, bg)

            gwait(j, u, b)
            sstart(j, u, b)
        return carry

    lax.fori_loop(0, NCHUNK // NIB, rnd, 0)
    for t in range(NBUF - GA):
        j = NCHUNK - (NBUF - GA) + t
        swait(j, j % NIB, j % NBUF)

    plsc.subcore_barrier()
    _copy_out(acc_sh, out_hbm, c, s)


@functools.cache
def _agg_call():
    return pl.kernel(
        _agg_body,
        out_type=jax.ShapeDtypeStruct((NC, NP, D), _F32),
        mesh=plsc.VectorSubcoreMesh(core_axis_name="c", subcore_axis_name="s"),
        scratch_types=[
            pltpu.VMEM((NIB, 2, K), jnp.int32),
            pltpu.VMEM((NBUF, K, D), _F32),
            pltpu.VMEM_SHARED((NP + NPAD, D), _F32),
            pltpu.SemaphoreType.DMA((NIB,)),
            pltpu.SemaphoreType.DMA((NBUF,)),
            pltpu.SemaphoreType.DMA((NBUF,)),
        ],
    )


BN = 1000
NBLK = N // BN


def _pre_body(x_ref, we_ref, be_ref, dg_ref, w0_ref, o_ref):
    h0 = _dot(x_ref[...], we_ref[...]) + be_ref[...]
    do = jnp.maximum(dg_ref[0], 1.0)
    o_ref[...] = _dot(h0 * lax.rsqrt(do), w0_ref[...])


def _mid_body(p_ref, dg_ref, b_ref, w_ref, o_ref):
    agg = p_ref[0] + p_ref[1]
    di = jnp.maximum(dg_ref[1], 1.0)
    do = jnp.maximum(dg_ref[0], 1.0)
    h = jnp.maximum(agg * lax.rsqrt(di) + b_ref[...], 0.0)
    hn = h * lax.rsqrt(do)
    o_ref[...] = _dot(hn, w_ref[...])


def _final_body(p_ref, dg_ref, b_ref, wo1_ref, bo1_ref, wo2_ref, bo2_ref,
                o_ref):
    agg = p_ref[0] + p_ref[1]
    di = jnp.maximum(dg_ref[1], 1.0)
    h = jnp.maximum(agg * lax.rsqrt(di) + b_ref[...], 0.0)
    t = jnp.maximum(_dot(h, wo1_ref[...]) + bo1_ref[...], 0.0)
    o_ref[...] = _dot(t, wo2_ref[...]) + bo2_ref[...]


def _row_spec(width):
    return pl.BlockSpec((BN, width), lambda i: (i, 0))


def _part_spec(width):
    return pl.BlockSpec((NC, BN, width), lambda i: (0, i, 0))


def _full_spec(r, cdim):
    return pl.BlockSpec((r, cdim), lambda i: (0, 0))


_pre_call = pl.pallas_call(
    _pre_body,
    grid=(NBLK,),
    in_specs=[_row_spec(D), _full_spec(D, D), _full_spec(1, D),
              _part_spec(1), _full_spec(D, D)],
    out_specs=_row_spec(D),
    out_shape=jax.ShapeDtypeStruct((N, D), _F32),
)

_mid_call = pl.pallas_call(
    _mid_body,
    grid=(NBLK,),
    in_specs=[_part_spec(D), _part_spec(1),
              _full_spec(1, D), _full_spec(D, D)],
    out_specs=_row_spec(D),
    out_shape=jax.ShapeDtypeStruct((N, D), _F32),
)

_final_call = pl.pallas_call(
    _final_body,
    grid=(NBLK,),
    in_specs=[_part_spec(D), _part_spec(1), _full_spec(1, D),
              _full_spec(D, D), _full_spec(1, D), _full_spec(D, 1),
              _full_spec(1, 1)],
    out_specs=_row_spec(1),
    out_shape=jax.ShapeDtypeStruct((N, 1), _F32),
)


def kernel(node_feats, edge_index, W_emb, b_emb, W0, b0, W1, b1, W2, b2,
           Wo1, bo1, Wo2, bo2):
    src = edge_index[0]
    dst = edge_index[1]
    npad = EP - E
    pad_dummy = (NP + (jnp.arange(npad, dtype=jnp.int32) % NPAD))
    pad_gather = (jnp.arange(npad, dtype=jnp.int32) * 131) % N
    src_p = jnp.concatenate([src, pad_gather])
    dst_p = jnp.concatenate([dst, pad_dummy])
    est = jnp.stack([src_p.reshape(NW, NCHUNK, K),
                     dst_p.reshape(NW, NCHUNK, K)], axis=2)
    eidx = jnp.concatenate([src, pad_dummy, dst, pad_dummy])

    deg = _deg_call()(eidx)[:, :, 0:1]
    m = _pre_call(node_feats, W_emb, b_emb.reshape(1, D), deg, W0)
    p = _agg_call()(m, est)
    m = _mid_call(p, deg, b0.reshape(1, D), W1)
    p = _agg_call()(m, est)
    m = _mid_call(p, deg, b1.reshape(1, D), W2)
    p = _agg_call()(m, est)
    out = _final_call(p, deg, b2.reshape(1, D), Wo1, bo1.reshape(1, D),
                      Wo2, bo2.reshape(1, 1))
    return out


# speedup vs baseline: 1.2134x; 1.0350x over previous
import functools

import jax
import jax.numpy as jnp
from jax import lax
from jax.experimental import pallas as pl
from jax.experimental.pallas import tpu as pltpu
from jax.experimental.pallas import tpu_sc as plsc

N = 10000
E = 320000
D = 128

NC = 2
NS = 16
NW = NC * NS
EW = E // NW
EP = 327680
EWP = EP // NW
K = 64
NCHUNK = EWP // K
NP = 10240
NPAD = 8
ZROWS = NP // NS
ZB = 128

_F32 = jnp.float32


def _dot(a, b):
    return jax.lax.dot_general(
        a, b, (((1,), (0,)), ((), ())),
        preferred_element_type=_F32, precision=lax.Precision.DEFAULT)


KD = 128
NCHUNK_D = (EP // NS) // KD
NIB_D = 8
IA_D = 4
WB = 64


def _fill_const(ref, nrows, vec16):
    def fill(r, carry):
        for j in range(D // 16):
            ref[r, pl.ds(j * 16, 16)] = vec16
        return carry

    lax.fori_loop(0, nrows, fill, 0)


def _zero_acc(zb_v, acc_sh, s, nrows):
    r0 = s * ZROWS
    for t in range(ZROWS // nrows):
        pltpu.sync_copy(zb_v, acc_sh.at[pl.ds(r0 + t * nrows, nrows)])


def _copy_out(acc_sh, out_hbm, c, s):
    r0 = s * ZROWS
    for t in range(ZROWS // ZB):
        pltpu.sync_copy(acc_sh.at[pl.ds(r0 + t * ZB, ZB)],
                        out_hbm.at[c, pl.ds(r0 + t * ZB, ZB)])


def _deg_body(eidx_hbm, out_hbm, idxr_v, ones_v, stage_v, wide_v, tbl_sh,
              isem, ssem):
    c = lax.axis_index("c")
    s = lax.axis_index("s")

    z16 = jnp.zeros((16,), _F32)
    for r in range(KD // 16):
        ones_v[pl.ds(r * 16, 16)] = z16
    r0 = s * ZROWS
    for t in range(ZROWS // KD):
        pltpu.sync_copy(ones_v, tbl_sh.at[pl.ds(r0 + t * KD, KD)])

    one16 = jnp.full((16,), 1.0, _F32)
    for r in range(KD // 16):
        ones_v[pl.ds(r * 16, 16)] = one16
    plsc.subcore_barrier()

    ebase = c * EP + s * (NCHUNK_D * KD)

    def istart(j, ib):
        pltpu.async_copy(eidx_hbm.at[pl.ds(ebase + j * KD, KD)],
                         idxr_v.at[ib], isem.at[ib])

    def iwait(j, ib):
        pltpu.make_async_copy(eidx_hbm.at[pl.ds(ebase + j * KD, KD)],
                              idxr_v.at[ib], isem.at[ib]).wait()

    def sstart(j, ib):
        pltpu.async_copy(ones_v, tbl_sh.at[idxr_v.at[ib]], ssem.at[ib],
                         add=True)

    def swait(j, ib):
        pltpu.make_async_copy(ones_v, tbl_sh.at[idxr_v.at[ib]],
                              ssem.at[ib]).wait()

    for j in range(IA_D):
        istart(j, j)

    def rnd(i, carry):
        for u in range(NIB_D):
            j = i * NIB_D + u
            ib4 = (u + IA_D) % NIB_D

            @pl.when(j >= IA_D)
            def _():
                swait(j - IA_D, ib4)

            @pl.when(j + IA_D < NCHUNK_D)
            def _():
                istart(j + IA_D, ib4)

            iwait(j, u)
            sstart(j, u)
        return carry

    lax.fori_loop(0, NCHUNK_D // NIB_D, rnd, 0)
    for t in range(IA_D):
        j = NCHUNK_D - IA_D + t
        swait(j, j % NIB_D)

    plsc.subcore_barrier()

    pltpu.sync_copy(tbl_sh.at[pl.ds(r0, ZROWS)], stage_v)

    for t in range(ZROWS // WB):
        def widen(g, carry):
            v = stage_v[pl.ds(t * WB + g * 16, 16)]
            for l in range(16):
                wide_v[g * 16 + l, pl.ds(0, 16)] = jnp.full((16,), v[l])
            return carry

        lax.fori_loop(0, WB // 16, widen, 0)
        pltpu.sync_copy(wide_v, out_hbm.at[c, pl.ds(r0 + t * WB, WB)])


@functools.cache
def _deg_call():
    return pl.kernel(
        _deg_body,
        out_type=jax.ShapeDtypeStruct((NC, NP, D), _F32),
        mesh=plsc.VectorSubcoreMesh(core_axis_name="c", subcore_axis_name="s"),
        scratch_types=[
            pltpu.VMEM((NIB_D, KD), jnp.int32),
            pltpu.VMEM((KD,), _F32),
            pltpu.VMEM((ZROWS,), _F32),
            pltpu.VMEM((WB, D), _F32),
            pltpu.VMEM_SHARED((NP + NPAD,), _F32),
            pltpu.SemaphoreType.DMA((NIB_D,)),
            pltpu.SemaphoreType.DMA((NIB_D,)),
        ],
    )


NBUF = 5
GA = 4
NIB = 10
IA = 6


def _agg_body(m_hbm, est_hbm, out_hbm, idxr_v, rows_v, acc_sh,
              isem, gsem, ssem):
    c = lax.axis_index("c")
    s = lax.axis_index("s")
    wid = s * NC + c

    _fill_const(rows_v.at[0], K, jnp.zeros((16,), _F32))
    _zero_acc(rows_v.at[0], acc_sh, s, K)
    plsc.subcore_barrier()

    def istart(j, ib):
        pltpu.async_copy(est_hbm.at[wid, j], idxr_v.at[ib], isem.at[ib])

    def iwait(j, ib):
        pltpu.make_async_copy(est_hbm.at[wid, j], idxr_v.at[ib],
                              isem.at[ib]).wait()

    def gstart(j, ib, b):
        pltpu.async_copy(m_hbm.at[idxr_v.at[ib, 0]], rows_v.at[b],
                         gsem.at[b])

    def gwait(j, ib, b):
        pltpu.make_async_copy(m_hbm.at[idxr_v.at[ib, 0]], rows_v.at[b],
                              gsem.at[b]).wait()

    def sstart(j, ib, b):
        pltpu.async_copy(rows_v.at[b], acc_sh.at[idxr_v.at[ib, 1]],
                         ssem.at[b], add=True)

    def swait(j, ib, b):
        pltpu.make_async_copy(rows_v.at[b], acc_sh.at[idxr_v.at[ib, 1]],
                              ssem.at[b]).wait()

    for j in range(IA):
        istart(j, j)
    for j in range(GA):
        iwait(j, j)
        gstart(j, j, j)

    def rnd(i, carry):
        for u in range(NIB):
            j = i * NIB + u
            b = u % NBUF
            ibg = (u + GA) % NIB
            bg = (u + GA) % NBUF
            ib4 = (u + IA) % NIB

            @pl.when(j + IA < NCHUNK)
            def _():
                istart(j + IA, ib4)

            @pl.when(j >= NBUF - GA)
            def _():
                swait(j - (NBUF - GA), (u - (NBUF - GA)) % NIB, bg)

            @pl.when(j + GA < NCHUNK)
            def _():
                iwait(j + GA, ibg)
                gstart(j + GA, ibg, bg)

            gwait(j, u, b)
            sstart(j, u, b)
        return carry

    lax.fori_loop(0, NCHUNK // NIB, rnd, 0)
    for t in range(NBUF - GA):
        j = NCHUNK - (NBUF - GA) + t
        swait(j, j % NIB, j % NBUF)

    plsc.subcore_barrier()
    _copy_out(acc_sh, out_hbm, c, s)


@functools.cache
def _agg_call():
    return pl.kernel(
        _agg_body,
        out_type=jax.ShapeDtypeStruct((NC, NP, D), _F32),
        mesh=plsc.VectorSubcoreMesh(core_axis_name="c", subcore_axis_name="s"),
        scratch_types=[
            pltpu.VMEM((NIB, 2, K), jnp.int32),
            pltpu.VMEM((NBUF, K, D), _F32),
            pltpu.VMEM_SHARED((NP + NPAD, D), _F32),
            pltpu.SemaphoreType.DMA((NIB,)),
            pltpu.SemaphoreType.DMA((NBUF,)),
            pltpu.SemaphoreType.DMA((NBUF,)),
        ],
    )


BN = 1000
NBLK = N // BN


def _pre_body(x_ref, we_ref, be_ref, dg_ref, w0_ref, o_ref):
    h0 = _dot(x_ref[...], we_ref[...]) + be_ref[...]
    do = jnp.maximum(dg_ref[0], 1.0)
    o_ref[...] = _dot(h0 * lax.rsqrt(do), w0_ref[...])


def _mid_body(p_ref, dg_ref, b_ref, w_ref, o_ref):
    agg = p_ref[0] + p_ref[1]
    di = jnp.maximum(dg_ref[1], 1.0)
    do = jnp.maximum(dg_ref[0], 1.0)
    h = jnp.maximum(agg * lax.rsqrt(di) + b_ref[...], 0.0)
    hn = h * lax.rsqrt(do)
    o_ref[...] = _dot(hn, w_ref[...])


def _final_body(p_ref, dg_ref, b_ref, wo1_ref, bo1_ref, wo2_ref, bo2_ref,
                o_ref):
    agg = p_ref[0] + p_ref[1]
    di = jnp.maximum(dg_ref[1], 1.0)
    h = jnp.maximum(agg * lax.rsqrt(di) + b_ref[...], 0.0)
    t = jnp.maximum(_dot(h, wo1_ref[...]) + bo1_ref[...], 0.0)
    o_ref[...] = _dot(t, wo2_ref[...]) + bo2_ref[...]


def _row_spec(width):
    return pl.BlockSpec((BN, width), lambda i: (i, 0))


def _part_spec(width):
    return pl.BlockSpec((NC, BN, width), lambda i: (0, i, 0))


def _full_spec(r, cdim):
    return pl.BlockSpec((r, cdim), lambda i: (0, 0))


_pre_call = pl.pallas_call(
    _pre_body,
    grid=(NBLK,),
    in_specs=[_row_spec(D), _full_spec(D, D), _full_spec(1, D),
              _part_spec(1), _full_spec(D, D)],
    out_specs=_row_spec(D),
    out_shape=jax.ShapeDtypeStruct((N, D), _F32),
)

_mid_call = pl.pallas_call(
    _mid_body,
    grid=(NBLK,),
    in_specs=[_part_spec(D), _part_spec(1),
              _full_spec(1, D), _full_spec(D, D)],
    out_specs=_row_spec(D),
    out_shape=jax.ShapeDtypeStruct((N, D), _F32),
)

_final_call = pl.pallas_call(
    _final_body,
    grid=(NBLK,),
    in_specs=[_part_spec(D), _part_spec(1), _full_spec(1, D),
              _full_spec(D, D), _full_spec(1, D), _full_spec(D, 1),
              _full_spec(1, 1)],
    out_specs=_row_spec(1),
    out_shape=jax.ShapeDtypeStruct((N, 1), _F32),
)


def kernel(node_feats, edge_index, W_emb, b_emb, W0, b0, W1, b1, W2, b2,
           Wo1, bo1, Wo2, bo2):
    src = edge_index[0]
    dst = edge_index[1]
    npad = EP - E
    pad_dummy = (NP + (jnp.arange(npad, dtype=jnp.int32) % NPAD))
    pad_gather = (jnp.arange(npad, dtype=jnp.int32) * 131) % N
    src_p = jnp.concatenate([src, pad_gather])
    dst_p = jnp.concatenate([dst, pad_dummy])
    est = jnp.stack([src_p.reshape(NW, NCHUNK, K),
                     dst_p.reshape(NW, NCHUNK, K)], axis=2)
    eidx = jnp.concatenate([src, pad_dummy, dst, pad_dummy])

    deg = _deg_call()(eidx)[:, :, 0:1]
    m = _pre_call(node_feats, W_emb, b_emb.reshape(1, D), deg, W0)
    p = _agg_call()(m, est)
    m = _mid_call(p, deg, b0.reshape(1, D), W1)
    p = _agg_call()(m, est)
    m = _mid_call(p, deg, b1.reshape(1, D), W2)
    p = _agg_call()(m, est)
    out = _final_call(p, deg, b2.reshape(1, D), Wo1, bo1.reshape(1, D),
                      Wo2, bo2.reshape(1, 1))
    return out
